# baseline (speedup 1.0000x reference)
import jax
import jax.numpy as jnp
from jax import lax
from jax.experimental import pallas as pl
from jax.experimental.pallas import tpu as pltpu

E = 64
K = 64
D = 1024
B_T = 4096

_ROWS = 512
_GRID1 = B_T // _ROWS


def _gate_body(x2d_ref, gw_ref, sv_ref, si_ref, cnt_ref, sc_ref):
    i = pl.program_id(0)
    s = jnp.dot(x2d_ref[...], gw_ref[...], preferred_element_type=jnp.float32)
    sc_ref[pl.ds(i * _ROWS, _ROWS), :] = s

    @pl.when(i == _GRID1 - 1)
    def _():
        riota = lax.broadcasted_iota(jnp.int32, (B_T, E), 0)

        def body(a, carry):
            sc, cnt = carry
            m = jnp.max(sc, axis=0)
            ismax = sc == m[None, :]
            idx = jnp.min(jnp.where(ismax, riota, B_T), axis=0)
            chosen = riota == idx[None, :]
            cnt = cnt + chosen.astype(jnp.float32)
            sc = jnp.where(chosen, -jnp.inf, sc)
            sv_ref[pl.ds(a, 1), :] = m.reshape(1, E)
            si_ref[pl.ds(a, 1), :] = idx.reshape(1, E)
            return sc, cnt

        init = (sc_ref[...], jnp.zeros((B_T, E), jnp.float32))
        _, cnt = lax.fori_loop(0, K, body, init)
        cnt_ref[...] = jnp.sum(cnt, axis=1, keepdims=True)


def _gate_topk(x2d, gw):
    return pl.pallas_call(
        _gate_body,
        grid=(_GRID1,),
        in_specs=[
            pl.BlockSpec((_ROWS, 2 * D), lambda i: (i, 0)),
            pl.BlockSpec((2 * D, E), lambda i: (0, 0)),
        ],
        out_specs=[
            pl.BlockSpec((K, E), lambda i: (0, 0)),
            pl.BlockSpec((K, E), lambda i: (0, 0)),
            pl.BlockSpec((B_T, 1), lambda i: (0, 0)),
        ],
        out_shape=[
            jax.ShapeDtypeStruct((K, E), jnp.float32),
            jax.ShapeDtypeStruct((K, E), jnp.int32),
            jax.ShapeDtypeStruct((B_T, 1), jnp.float32),
        ],
        scratch_shapes=[pltpu.VMEM((B_T, E), jnp.float32)],
    )(x2d, gw)


def _expert_body(xgr_ref, xgi_ref, w_ref, s_ref, y_ref):
    xc = jnp.concatenate([xgr_ref[...], xgi_ref[...]], axis=0)
    w = w_ref[...]
    ab = jnp.dot(xc, w, preferred_element_type=jnp.float32)
    a = ab[:K]
    b = ab[K:]
    rm1 = pltpu.roll(b, 2 * D - 1, axis=1)
    r1 = pltpu.roll(b, 1, axis=1)
    lane = lax.broadcasted_iota(jnp.int32, (K, 2 * D), 1)
    bswap = jnp.where(lane % 2 == 0, -rm1, r1)
    y_ref[...] = (a + bswap) * s_ref[...]


def _experts(xgr, xgi, w3, sflat):
    return pl.pallas_call(
        _expert_body,
        grid=(E,),
        in_specs=[
            pl.BlockSpec((K, D), lambda a: (a, 0)),
            pl.BlockSpec((K, D), lambda a: (a, 0)),
            pl.BlockSpec((D, 2 * D), lambda a: (a, 0)),
            pl.BlockSpec((K, 1), lambda a: (a, 0)),
        ],
        out_specs=pl.BlockSpec((K, 2 * D), lambda a: (a, 0)),
        out_shape=jax.ShapeDtypeStruct((B_T, 2 * D), jnp.float32),
    )(xgr, xgi, w3, sflat)


_INV_SQRT2 = 0.7071067811865476


def _gelu_exact(v):
    return 0.5 * v * (1.0 + lax.erf(v * _INV_SQRT2))


def _finalize_body(out_ref, cnt_ref, bias_ref, res_ref):
    cnt = jnp.clip(cnt_ref[...], 1.0, None)
    res_ref[...] = _gelu_exact(out_ref[...] / cnt + bias_ref[...])


def _finalize(out2d, counts, bias_int):
    return pl.pallas_call(
        _finalize_body,
        grid=(_GRID1,),
        in_specs=[
            pl.BlockSpec((_ROWS, 2 * D), lambda i: (i, 0)),
            pl.BlockSpec((_ROWS, 1), lambda i: (i, 0)),
            pl.BlockSpec((1, 2 * D), lambda i: (0, 0)),
        ],
        out_specs=pl.BlockSpec((_ROWS, 2 * D), lambda i: (i, 0)),
        out_shape=jax.ShapeDtypeStruct((B_T, 2 * D), jnp.float32),
    )(out2d, counts, bias_int)


def kernel(x, gate_weights, experts_weight, act_bias):
    x2d = x.reshape(B_T, 2 * D)
    xrb = x[:, :, 0].astype(jnp.bfloat16)
    xib = x[:, :, 1].astype(jnp.bfloat16)
    w3 = experts_weight.reshape(E * D, 2 * D).astype(jnp.bfloat16)

    sv, si, counts = _gate_topk(x2d, gate_weights)
    topk_scores = sv.T
    topk_indices = si.T
    flat = si.reshape(-1)

    xgr = xrb[flat]
    xgi = xib[flat]

    y_all = _experts(xgr, xgi, w3, sv.reshape(B_T, 1))

    out2d = jnp.zeros((B_T, 2 * D), jnp.float32).at[flat].add(y_all)

    res2d = _finalize(out2d, counts, jnp.repeat(act_bias, 2).reshape(1, 2 * D))
    res = res2d.reshape(B_T, D, 2)
    return (res, topk_indices, topk_scores, counts.reshape(B_T, 1, 1))
